# baseline (speedup 1.0000x reference)
import functools
import jax
import jax.numpy as jnp
import numpy as np
from jax.experimental import pallas as pl
from jax.experimental.pallas import tpu as pltpu

_B, _N, _UP = 16, 1024, 4
_SA_CFG = [
    (1024, 0.05, 32),
    (512, 0.1, 32),
    (256, 0.2, 32),
    (128, 0.3, 32),
]


_BF = jnp.bfloat16
_HI = jax.lax.Precision.HIGHEST


def _nn_dot(a, b, precision=None):
    return jax.lax.dot_general(a, b, (((1,), (0,)), ((), ())),
                               precision=precision,
                               preferred_element_type=jnp.float32)


_NPOINTS = [1024, 512, 256, 128]


def _fps_level(X, Y, Z, npoint, o_ref, selord_ref):
    n = X.shape[1]
    iota = jax.lax.broadcasted_iota(jnp.int32, (_B, n), 1)

    def body(i, state):
        dmin, far, selord = state
        mask = iota == far
        maskf = mask.astype(jnp.float32)
        cx = jnp.sum(X * maskf, axis=-1, keepdims=True)
        cy = jnp.sum(Y * maskf, axis=-1, keepdims=True)
        cz = jnp.sum(Z * maskf, axis=-1, keepdims=True)
        selord = jnp.where(mask, i, selord)
        d = (X - cx) ** 2 + (Y - cy) ** 2 + (Z - cz) ** 2
        dmin = jnp.minimum(dmin, d)
        far = jnp.argmax(dmin, axis=-1)[:, None].astype(jnp.int32)
        return (dmin, far, selord)

    selord_ref[:, 0:n] = jnp.full((_B, n), n, jnp.int32)
    init = (jnp.full((_B, n), 1e10, jnp.float32),
            jnp.zeros((_B, 1), jnp.int32),
            selord_ref[:, 0:n])
    _, _, selord = jax.lax.fori_loop(0, npoint, body, init)
    selord_ref[:, 0:n] = selord

    iota_s = jax.lax.broadcasted_iota(jnp.int32, (npoint, n), 0)
    for b in range(_B):
        row = selord_ref[pl.ds(b, 1), 0:n]
        ohT = (iota_s == row).astype(jnp.float32)
        Cb = jnp.concatenate([X[b:b + 1], Y[b:b + 1], Z[b:b + 1]], axis=0)
        res = jax.lax.dot_general(Cb, ohT, (((1,), (1,)), ((), ())),
                                  precision=jax.lax.Precision.HIGHEST,
                                  preferred_element_type=jnp.float32)
        o_ref[:, pl.ds(b, 1), :] = res[:, None, :]


def _fps_chain_kernel(xt_ref, o1_ref, o2_ref, o3_ref, o4_ref, selord_ref):
    X, Y, Z = xt_ref[0], xt_ref[1], xt_ref[2]
    _fps_level(X, Y, Z, _NPOINTS[0], o1_ref, selord_ref)
    X, Y, Z = o1_ref[0], o1_ref[1], o1_ref[2]
    _fps_level(X, Y, Z, _NPOINTS[1], o2_ref, selord_ref)
    X, Y, Z = o2_ref[0], o2_ref[1], o2_ref[2]
    _fps_level(X, Y, Z, _NPOINTS[2], o3_ref, selord_ref)
    X, Y, Z = o3_ref[0], o3_ref[1], o3_ref[2]
    _fps_level(X, Y, Z, _NPOINTS[3], o4_ref, selord_ref)


def _fps_chain(x):
    xt = x.transpose(2, 0, 1)
    outs = pl.pallas_call(
        _fps_chain_kernel,
        out_shape=[jax.ShapeDtypeStruct((3, _B, s), jnp.float32)
                   for s in _NPOINTS],
        scratch_shapes=[pltpu.VMEM((_B, _N), jnp.int32)],
    )(xt)
    return [o.transpose(1, 2, 0) for o in outs]


def _sa_kernel_body(radius, k, has_feats, *refs):
    if has_feats:
        (qn_ref, pn_ref, pT_ref, f_ref, w1x_ref, w1f_ref, b1_ref,
         w2_ref, b2_ref, w3_ref, b3_ref, out_ref, acc_ref) = refs
    else:
        (qn_ref, pn_ref, pT_ref, w1x_ref, b1_ref,
         w2_ref, b2_ref, w3_ref, b3_ref, out_ref, acc_ref) = refs
    qn = qn_ref[0]
    pn = pn_ref[0]
    pT = pT_ref[0]
    s, n = qn.shape[0], pn.shape[0]
    r2 = radius * radius

    qs = jnp.sum(qn * qn, axis=-1, keepdims=True)
    ps = jnp.sum(pT * pT, axis=0, keepdims=True)
    cross = _nn_dot(qn.astype(_BF), pT.astype(_BF))
    D = (qs + ps) - 2.0 * cross

    maskb = D <= r2
    maskf = jnp.where(maskb, 1.0, 0.0)
    ltri = (jax.lax.broadcasted_iota(jnp.int32, (n, n), 0)
            < jax.lax.broadcasted_iota(jnp.int32, (n, n), 1)).astype(_BF)
    rank = _nn_dot(maskf.astype(_BF), ltri)
    cnt = jnp.sum(maskf, axis=-1, keepdims=True)
    empty = cnt == 0.0
    iota_n = jax.lax.broadcasted_iota(jnp.int32, (s, n), 1)
    lastcol = iota_n == (n - 1)

    if has_feats:
        pcat = jnp.concatenate([pn.astype(_BF), f_ref[0].astype(_BF)], axis=-1)
    else:
        pcat = pn.astype(_BF)
    w1x = w1x_ref[...].astype(_BF)
    w1f = w1f_ref[...].astype(_BF) if has_feats else None
    w2 = w2_ref[...].astype(_BF)
    w3 = w3_ref[...].astype(_BF)
    b1, b2, b3 = b1_ref[...], b2_ref[...], b3_ref[...]

    acc_ref[...] = jnp.zeros_like(acc_ref)

    def slot(kk, _):
        kkf = kk.astype(jnp.float32)
        v = jnp.where(cnt > kkf, kkf, 0.0)
        sel = (maskb & (rank == v)) | (empty & lastcol)
        A = jnp.where(sel, 1.0, 0.0).astype(_BF)
        G = _nn_dot(A, pcat)
        off = G[:, 0:3] - qn
        h = _nn_dot(off.astype(_BF), w1x)
        if has_feats:
            h = h + _nn_dot(G[:, 3:].astype(_BF), w1f)
        h = jnp.maximum(h + b1, 0.0)
        h = jnp.maximum(_nn_dot(h.astype(_BF), w2) + b2, 0.0)
        h = jnp.maximum(_nn_dot(h.astype(_BF), w3) + b3, 0.0)
        acc_ref[...] = jnp.maximum(acc_ref[...], h)
        return 0

    kmax = jnp.clip(jnp.max(cnt).astype(jnp.int32), 1, k)
    jax.lax.fori_loop(0, kmax, slot, 0)
    out_ref[0] = acc_ref[...]


def _sa_level(new_xyz, xyz, feats, radius, k, layers):
    s, n = new_xyz.shape[1], xyz.shape[1]
    has_feats = feats is not None
    (w1, b1), (w2, b2), (w3, b3) = layers
    c1, c2, c3 = w1.shape[1], w2.shape[1], w3.shape[1]
    xT = xyz.transpose(0, 2, 1)

    args = [new_xyz, xyz, xT]
    specs = [
        pl.BlockSpec((1, s, 3), lambda b: (b, 0, 0)),
        pl.BlockSpec((1, n, 3), lambda b: (b, 0, 0)),
        pl.BlockSpec((1, 3, n), lambda b: (b, 0, 0)),
    ]
    if has_feats:
        c = feats.shape[2]
        args.append(feats)
        specs.append(pl.BlockSpec((1, n, c), lambda b: (b, 0, 0)))
        w_args = [w1[0:3], w1[3:], b1[None, :], w2, b2[None, :], w3, b3[None, :]]
    else:
        w_args = [w1, b1[None, :], w2, b2[None, :], w3, b3[None, :]]
    args += w_args
    specs += [pl.BlockSpec(a.shape, (lambda b, r=len(a.shape): (0,) * r))
              for a in w_args]

    body = functools.partial(_sa_kernel_body, radius, k, has_feats)
    out = pl.pallas_call(
        body,
        grid=(_B,),
        in_specs=specs,
        out_specs=pl.BlockSpec((1, s, c3), lambda b: (b, 0, 0)),
        out_shape=jax.ShapeDtypeStruct((_B, s, c3), jnp.float32),
        scratch_shapes=[pltpu.VMEM((s, c3), jnp.float32)],
    )(*args)
    return out


def _fp_kernel_body(qn_ref, pT_ref, f_ref, w_ref, b_ref, out_ref):
    qn = qn_ref[0]
    pT = pT_ref[0]
    F = f_ref[0]
    m, s2 = qn.shape[0], pT.shape[1]

    qs = jnp.sum(qn * qn, axis=-1, keepdims=True)
    ps = jnp.sum(pT * pT, axis=0, keepdims=True)
    cross = _nn_dot(qn.astype(_BF), pT.astype(_BF))
    D = (qs + ps) - 2.0 * cross

    iota = jax.lax.broadcasted_iota(jnp.int32, (m, s2), 1)
    Dw = D
    ds, ohs = [], []
    for _ in range(3):
        mv = jnp.min(Dw, axis=-1, keepdims=True)
        j = jnp.min(jnp.where(Dw == mv, iota, s2), axis=-1, keepdims=True)
        oh = (iota == j)
        ds.append(mv)
        ohs.append(jnp.where(oh, 1.0, 0.0))
        Dw = jnp.where(oh, jnp.float32(3e38), Dw)

    ws = [1.0 / (jnp.maximum(d, 0.0) + 1e-8) for d in ds]
    tot = ws[0] + ws[1] + ws[2]
    h = None
    for w_i, oh_i in zip(ws, ohs):
        g = _nn_dot(oh_i, F, precision=_HI)
        term = g * (w_i / tot)
        h = term if h is None else h + term
    h = jnp.maximum(_nn_dot(h.astype(_BF), w_ref[...].astype(_BF))
                    + b_ref[...], 0.0)
    out_ref[0] = h


def _fp_level(xyz1, xyz2, points2, layers):
    m, s2, c = xyz1.shape[1], xyz2.shape[1], points2.shape[2]
    (w, b), = layers
    co = w.shape[1]
    pT = xyz2.transpose(0, 2, 1)
    out = pl.pallas_call(
        _fp_kernel_body,
        grid=(_B,),
        in_specs=[
            pl.BlockSpec((1, m, 3), lambda bb: (bb, 0, 0)),
            pl.BlockSpec((1, 3, s2), lambda bb: (bb, 0, 0)),
            pl.BlockSpec((1, s2, c), lambda bb: (bb, 0, 0)),
            pl.BlockSpec(w.shape, lambda bb: (0, 0)),
            pl.BlockSpec((1, co), lambda bb: (0, 0)),
        ],
        out_specs=pl.BlockSpec((1, m, co), lambda bb: (bb, 0, 0)),
        out_shape=jax.ShapeDtypeStruct((_B, m, co), jnp.float32),
    )(xyz1, pT, points2, w, b[None, :])
    return out


def _final_stage_kernel(concat_ref, w1_ref, b1_ref, w2_ref, b2_ref,
                        cw1_ref, cb1_ref, cw2_ref, cb2_ref, out_ref):
    h = concat_ref[0]
    h = jnp.maximum(_nn_dot(h.astype(_BF), w1_ref[0].astype(_BF))
                    + b1_ref[0], 0.0)
    h = jnp.maximum(_nn_dot(h.astype(_BF), w2_ref[0].astype(_BF))
                    + b2_ref[0], 0.0)
    h1 = _nn_dot(h.astype(_BF), cw1_ref[...].astype(_BF)) + cb1_ref[...]
    h1 = jnp.where(h1 > 0, h1, 0.2 * h1)
    out_ref[0, 0] = (_nn_dot(h1.astype(_BF), cw2_ref[...].astype(_BF))
                     + cb2_ref[...])


def _final_stage(concat, params):
    up_w1 = jnp.stack([params['up'][u][0][0] for u in range(_UP)])
    up_b1 = jnp.stack([params['up'][u][0][1] for u in range(_UP)])[:, None, :]
    up_w2 = jnp.stack([params['up'][u][1][0] for u in range(_UP)])
    up_b2 = jnp.stack([params['up'][u][1][1] for u in range(_UP)])[:, None, :]
    cw1, cb1 = params['conv1']
    cw2, cb2 = params['conv2']
    cb1 = cb1[None, :]
    cb2 = cb2[None, :]
    c_in = concat.shape[-1]

    out = pl.pallas_call(
        _final_stage_kernel,
        grid=(_UP, _B),
        in_specs=[
            pl.BlockSpec((1, _N, c_in), lambda u, b: (b, 0, 0)),
            pl.BlockSpec((1, c_in, 256), lambda u, b: (u, 0, 0)),
            pl.BlockSpec((1, 1, 256), lambda u, b: (u, 0, 0)),
            pl.BlockSpec((1, 256, 128), lambda u, b: (u, 0, 0)),
            pl.BlockSpec((1, 1, 128), lambda u, b: (u, 0, 0)),
            pl.BlockSpec((128, 64), lambda u, b: (0, 0)),
            pl.BlockSpec((1, 64), lambda u, b: (0, 0)),
            pl.BlockSpec((64, 3), lambda u, b: (0, 0)),
            pl.BlockSpec((1, 3), lambda u, b: (0, 0)),
        ],
        out_specs=pl.BlockSpec((1, 1, _N, 3), lambda u, b: (u, b, 0, 0)),
        out_shape=jax.ShapeDtypeStruct((_UP, _B, _N, 3), jnp.float32),
    )(concat, up_w1, up_b1, up_w2, up_b2, cw1, cb1, cw2, cb2)
    return out.transpose(1, 0, 2, 3).reshape(_B, _UP * _N, 3)


def kernel(x, params):
    l0_xyz = x[:, :, 0:3]
    xyzs = [l0_xyz] + _fps_chain(l0_xyz)
    feats = [None]
    cur_pts = None
    for li, ((npoint, radius, k), layers) in enumerate(zip(_SA_CFG, params['sa'])):
        cur_pts = _sa_level(xyzs[li + 1], xyzs[li], cur_pts, radius, k, layers)
        feats.append(cur_pts)
    up_l4 = _fp_level(l0_xyz, xyzs[4], feats[4], params['fp'][0])
    up_l3 = _fp_level(l0_xyz, xyzs[3], feats[3], params['fp'][1])
    up_l2 = _fp_level(l0_xyz, xyzs[2], feats[2], params['fp'][2])
    concat = jnp.concatenate([up_l4, up_l3, up_l2, feats[1], l0_xyz], axis=-1)
    return _final_stage(concat, params)


# speedup vs baseline: 1.0302x; 1.0302x over previous
import functools
import jax
import jax.numpy as jnp
import numpy as np
from jax.experimental import pallas as pl
from jax.experimental.pallas import tpu as pltpu

_B, _N, _UP = 16, 1024, 4
_SA_CFG = [
    (1024, 0.05, 32),
    (512, 0.1, 32),
    (256, 0.2, 32),
    (128, 0.3, 32),
]


_BF = jnp.bfloat16
_HI = jax.lax.Precision.HIGHEST


def _nn_dot(a, b, precision=None):
    return jax.lax.dot_general(a, b, (((1,), (0,)), ((), ())),
                               precision=precision,
                               preferred_element_type=jnp.float32)


_NPOINTS = [1024, 512, 256, 128]


def _fps_level(X, Y, Z, npoint, o_ref, selord_ref):
    n = X.shape[1]
    iota = jax.lax.broadcasted_iota(jnp.int32, (_B, n), 1)

    def body(i, state):
        dmin, far, selord = state
        mask = iota == far
        maskf = mask.astype(jnp.float32)
        cx = jnp.sum(X * maskf, axis=-1, keepdims=True)
        cy = jnp.sum(Y * maskf, axis=-1, keepdims=True)
        cz = jnp.sum(Z * maskf, axis=-1, keepdims=True)
        selord = jnp.where(mask, i, selord)
        d = (X - cx) ** 2 + (Y - cy) ** 2 + (Z - cz) ** 2
        dmin = jnp.minimum(dmin, d)
        far = jnp.argmax(dmin, axis=-1)[:, None].astype(jnp.int32)
        return (dmin, far, selord)

    selord_ref[:, 0:n] = jnp.full((_B, n), n, jnp.int32)
    init = (jnp.full((_B, n), 1e10, jnp.float32),
            jnp.zeros((_B, 1), jnp.int32),
            selord_ref[:, 0:n])
    _, _, selord = jax.lax.fori_loop(0, npoint, body, init)
    selord_ref[:, 0:n] = selord

    iota_s = jax.lax.broadcasted_iota(jnp.int32, (npoint, n), 0)
    for b in range(_B):
        row = selord_ref[pl.ds(b, 1), 0:n]
        ohT = (iota_s == row).astype(jnp.float32)
        Cb = jnp.concatenate([X[b:b + 1], Y[b:b + 1], Z[b:b + 1]], axis=0)
        res = jax.lax.dot_general(Cb, ohT, (((1,), (1,)), ((), ())),
                                  precision=jax.lax.Precision.HIGHEST,
                                  preferred_element_type=jnp.float32)
        o_ref[:, pl.ds(b, 1), :] = res[:, None, :]


def _fps_chain_kernel(xt_ref, o1_ref, o2_ref, o3_ref, o4_ref, selord_ref):
    X, Y, Z = xt_ref[0], xt_ref[1], xt_ref[2]
    _fps_level(X, Y, Z, _NPOINTS[0], o1_ref, selord_ref)
    X, Y, Z = o1_ref[0], o1_ref[1], o1_ref[2]
    _fps_level(X, Y, Z, _NPOINTS[1], o2_ref, selord_ref)
    X, Y, Z = o2_ref[0], o2_ref[1], o2_ref[2]
    _fps_level(X, Y, Z, _NPOINTS[2], o3_ref, selord_ref)
    X, Y, Z = o3_ref[0], o3_ref[1], o3_ref[2]
    _fps_level(X, Y, Z, _NPOINTS[3], o4_ref, selord_ref)


def _fps_chain(x):
    xt = x.transpose(2, 0, 1)
    outs = pl.pallas_call(
        _fps_chain_kernel,
        out_shape=[jax.ShapeDtypeStruct((3, _B, s), jnp.float32)
                   for s in _NPOINTS],
        scratch_shapes=[pltpu.VMEM((_B, _N), jnp.int32)],
    )(xt)
    return [o.transpose(1, 2, 0) for o in outs]


def _sa_kernel_body(radius, k, has_feats, *refs):
    if has_feats:
        (qn_ref, pn_ref, pT_ref, f_ref, w1x_ref, w1f_ref, b1_ref,
         w2_ref, b2_ref, w3_ref, b3_ref, out_ref, acc_ref) = refs
    else:
        (qn_ref, pn_ref, pT_ref, w1x_ref, b1_ref,
         w2_ref, b2_ref, w3_ref, b3_ref, out_ref, acc_ref) = refs
    qn = qn_ref[0]
    pn = pn_ref[0]
    pT = pT_ref[0]
    s, n = qn.shape[0], pn.shape[0]
    r2 = radius * radius

    qs = jnp.sum(qn * qn, axis=-1, keepdims=True)
    ps = jnp.sum(pT * pT, axis=0, keepdims=True)
    cross = _nn_dot(qn.astype(_BF), pT.astype(_BF))
    D = (qs + ps) - 2.0 * cross

    maskb = D <= r2
    maskf = jnp.where(maskb, 1.0, 0.0)
    ltri = (jax.lax.broadcasted_iota(jnp.int32, (n, n), 0)
            < jax.lax.broadcasted_iota(jnp.int32, (n, n), 1)).astype(_BF)
    rank = _nn_dot(maskf.astype(_BF), ltri)
    cnt = jnp.sum(maskf, axis=-1, keepdims=True)
    empty = cnt == 0.0
    iota_n = jax.lax.broadcasted_iota(jnp.int32, (s, n), 1)
    lastcol = iota_n == (n - 1)

    if has_feats:
        pcat = jnp.concatenate([pn.astype(_BF), f_ref[0].astype(_BF)], axis=-1)
    else:
        pcat = pn.astype(_BF)
    w1x = w1x_ref[...].astype(_BF)
    w1f = w1f_ref[...].astype(_BF) if has_feats else None
    w2 = w2_ref[...].astype(_BF)
    w3 = w3_ref[...].astype(_BF)
    b1, b2, b3 = b1_ref[...], b2_ref[...], b3_ref[...]

    acc_ref[...] = jnp.zeros_like(acc_ref)

    def slot(kk, _):
        kkf = kk.astype(jnp.float32)
        v = jnp.where(cnt > kkf, kkf, 0.0)
        sel = (maskb & (rank == v)) | (empty & lastcol)
        A = jnp.where(sel, 1.0, 0.0).astype(_BF)
        G = _nn_dot(A, pcat)
        off = G[:, 0:3] - qn
        h = _nn_dot(off.astype(_BF), w1x)
        if has_feats:
            h = h + _nn_dot(G[:, 3:].astype(_BF), w1f)
        h = jnp.maximum(h + b1, 0.0)
        h = jnp.maximum(_nn_dot(h.astype(_BF), w2) + b2, 0.0)
        h = jnp.maximum(_nn_dot(h.astype(_BF), w3) + b3, 0.0)
        acc_ref[...] = jnp.maximum(acc_ref[...], h)
        return 0

    kmax = jnp.clip(jnp.max(cnt).astype(jnp.int32), 1, k)
    jax.lax.fori_loop(0, kmax, slot, 0)
    out_ref[0] = acc_ref[...]


def _sa_level(new_xyz, xyz, feats, radius, k, layers):
    s, n = new_xyz.shape[1], xyz.shape[1]
    has_feats = feats is not None
    (w1, b1), (w2, b2), (w3, b3) = layers
    c1, c2, c3 = w1.shape[1], w2.shape[1], w3.shape[1]
    xT = xyz.transpose(0, 2, 1)

    args = [new_xyz, xyz, xT]
    specs = [
        pl.BlockSpec((1, s, 3), lambda b: (b, 0, 0)),
        pl.BlockSpec((1, n, 3), lambda b: (b, 0, 0)),
        pl.BlockSpec((1, 3, n), lambda b: (b, 0, 0)),
    ]
    if has_feats:
        c = feats.shape[2]
        args.append(feats)
        specs.append(pl.BlockSpec((1, n, c), lambda b: (b, 0, 0)))
        w_args = [w1[0:3], w1[3:], b1[None, :], w2, b2[None, :], w3, b3[None, :]]
    else:
        w_args = [w1, b1[None, :], w2, b2[None, :], w3, b3[None, :]]
    args += w_args
    specs += [pl.BlockSpec(a.shape, (lambda b, r=len(a.shape): (0,) * r))
              for a in w_args]

    body = functools.partial(_sa_kernel_body, radius, k, has_feats)
    out = pl.pallas_call(
        body,
        grid=(_B,),
        in_specs=specs,
        out_specs=pl.BlockSpec((1, s, c3), lambda b: (b, 0, 0)),
        out_shape=jax.ShapeDtypeStruct((_B, s, c3), jnp.float32),
        scratch_shapes=[pltpu.VMEM((s, c3), jnp.float32)],
    )(*args)
    return out


def _fp_compute(qn, qs, pT, F, w, b):
    m, s2 = qn.shape[0], pT.shape[1]
    ps = jnp.sum(pT * pT, axis=0, keepdims=True)
    cross = _nn_dot(qn.astype(_BF), pT.astype(_BF))
    D = (qs + ps) - 2.0 * cross

    iota = jax.lax.broadcasted_iota(jnp.int32, (m, s2), 1)
    Dw = D
    ds, ohs = [], []
    for _ in range(3):
        mv = jnp.min(Dw, axis=-1, keepdims=True)
        j = jnp.min(jnp.where(Dw == mv, iota, s2), axis=-1, keepdims=True)
        oh = (iota == j)
        ds.append(mv)
        ohs.append(jnp.where(oh, 1.0, 0.0))
        Dw = jnp.where(oh, jnp.float32(3e38), Dw)

    ws = [1.0 / (jnp.maximum(d, 0.0) + 1e-8) for d in ds]
    tot = ws[0] + ws[1] + ws[2]
    h = None
    for w_i, oh_i in zip(ws, ohs):
        g = _nn_dot(oh_i, F, precision=_HI)
        term = g * (w_i / tot)
        h = term if h is None else h + term
    return jnp.maximum(_nn_dot(h.astype(_BF), w.astype(_BF)) + b, 0.0)


def _tail_kernel_body(qn_ref, pTA_ref, fA_ref, pTB_ref, fB_ref, pTC_ref,
                      fC_ref, f1_ref, wA_ref, bA_ref, wB_ref, bB_ref,
                      wC_ref, bC_ref, uw1_ref, ub1_ref, uw2_ref, ub2_ref,
                      cw1_ref, cb1_ref, cw2_ref, cb2_ref, out_ref):
    qn = qn_ref[0]
    qs = jnp.sum(qn * qn, axis=-1, keepdims=True)
    upA = _fp_compute(qn, qs, pTA_ref[0], fA_ref[0], wA_ref[...], bA_ref[...])
    upB = _fp_compute(qn, qs, pTB_ref[0], fB_ref[0], wB_ref[...], bB_ref[...])
    upC = _fp_compute(qn, qs, pTC_ref[0], fC_ref[0], wC_ref[...], bC_ref[...])
    cc = jnp.concatenate([upA, upB, upC, f1_ref[0], qn], axis=-1)
    cc = cc.astype(_BF)
    cw1 = cw1_ref[...].astype(_BF)
    cw2 = cw2_ref[...].astype(_BF)
    for u in range(_UP):
        h = jnp.maximum(_nn_dot(cc, uw1_ref[u].astype(_BF)) + ub1_ref[u], 0.0)
        h = jnp.maximum(_nn_dot(h.astype(_BF), uw2_ref[u].astype(_BF))
                        + ub2_ref[u], 0.0)
        h1 = _nn_dot(h.astype(_BF), cw1) + cb1_ref[...]
        h1 = jnp.where(h1 > 0, h1, 0.2 * h1)
        out_ref[0, pl.ds(u * _N, _N), :] = (_nn_dot(h1.astype(_BF), cw2)
                                            + cb2_ref[...])


def _tail_stage(l0_xyz, xyzs, feats, params):
    pTA = xyzs[4].transpose(0, 2, 1)
    pTB = xyzs[3].transpose(0, 2, 1)
    pTC = xyzs[2].transpose(0, 2, 1)
    (wA, bA), = params['fp'][0]
    (wB, bB), = params['fp'][1]
    (wC, bC), = params['fp'][2]
    uw1 = jnp.stack([params['up'][u][0][0] for u in range(_UP)])
    ub1 = jnp.stack([params['up'][u][0][1] for u in range(_UP)])[:, None, :]
    uw2 = jnp.stack([params['up'][u][1][0] for u in range(_UP)])
    ub2 = jnp.stack([params['up'][u][1][1] for u in range(_UP)])[:, None, :]
    cw1, cb1 = params['conv1']
    cw2, cb2 = params['conv2']

    args = [l0_xyz, pTA, feats[4], pTB, feats[3], pTC, feats[2], feats[1],
            wA, bA[None, :], wB, bB[None, :], wC, bC[None, :],
            uw1, ub1, uw2, ub2, cw1, cb1[None, :], cw2, cb2[None, :]]
    specs = []
    for i, a in enumerate(args):
        if i < 8:
            shp = (1,) + a.shape[1:]
            specs.append(pl.BlockSpec(shp, (lambda bb, r=len(a.shape) - 1:
                                            (bb,) + (0,) * r)))
        else:
            specs.append(pl.BlockSpec(a.shape,
                                      (lambda bb, r=len(a.shape): (0,) * r)))

    out = pl.pallas_call(
        _tail_kernel_body,
        grid=(_B,),
        in_specs=specs,
        out_specs=pl.BlockSpec((1, _UP * _N, 3), lambda bb: (bb, 0, 0)),
        out_shape=jax.ShapeDtypeStruct((_B, _UP * _N, 3), jnp.float32),
    )(*args)
    return out


def kernel(x, params):
    l0_xyz = x[:, :, 0:3]
    xyzs = [l0_xyz] + _fps_chain(l0_xyz)
    feats = [None]
    cur_pts = None
    for li, ((npoint, radius, k), layers) in enumerate(zip(_SA_CFG, params['sa'])):
        cur_pts = _sa_level(xyzs[li + 1], xyzs[li], cur_pts, radius, k, layers)
        feats.append(cur_pts)
    return _tail_stage(l0_xyz, xyzs, feats, params)
